# baseline (speedup 1.0000x reference)
import functools

import jax
import jax.numpy as jnp
from jax import lax
from jax.experimental import pallas as pl
from jax.experimental.pallas import tpu as pltpu
from jax.experimental.pallas import tpu_sc as plsc

B, N, K, CIN, COUT, KS = 2, 4096, 16, 128, 128, 5
M = B * N
MK = M * K
RADIUS, TAU, EPS = 1.0, 1.0, 1e-5
_F32 = jnp.float32
_BF16 = jnp.bfloat16

_NC, _NS = 2, 16
_NW = _NC * _NS
_PW = MK // _NW
_CH = 128
_NCH = _PW // _CH


def _make_sc_gather(D, dtype):
  mesh = plsc.VectorSubcoreMesh(core_axis_name="c", subcore_axis_name="s")

  @functools.partial(
      pl.kernel,
      out_type=jax.ShapeDtypeStruct((MK, D), dtype),
      mesh=mesh,
      scratch_types=[
          pltpu.VMEM((_PW,), jnp.int32),
          pltpu.VMEM((_CH, D), dtype),
          pltpu.VMEM((_CH, D), dtype),
          pltpu.SemaphoreType.DMA,
          pltpu.SemaphoreType.DMA,
      ],
  )
  def gather(table_hbm, idx_hbm, out_hbm, idx_v, buf0, buf1, sem0, sem1):
    wid = lax.axis_index("s") * _NC + lax.axis_index("c")
    base = wid * _PW
    pltpu.sync_copy(idx_hbm.at[pl.ds(base, _PW)], idx_v)
    bufs = (buf0, buf1)
    sems = (sem0, sem1)

    def start(c, b):
      pltpu.make_async_copy(
          table_hbm.at[idx_v.at[pl.ds(c * _CH, _CH)]], bufs[b], sems[b]
      ).start()

    def wait(b):
      pltpu.make_async_copy(
          table_hbm.at[idx_v.at[pl.ds(0, _CH)]], bufs[b], sems[b]
      ).wait()

    def store(c, b):
      pltpu.sync_copy(bufs[b], out_hbm.at[pl.ds(base + c * _CH, _CH)])

    start(0, 0)
    start(1, 1)

    def body(j, carry):
      c0 = 2 * j
      wait(0)
      store(c0, 0)
      start(c0 + 2, 0)
      wait(1)
      store(c0 + 1, 1)
      start(c0 + 3, 1)
      return carry

    lax.fori_loop(0, _NCH // 2 - 1, body, 0)
    wait(0)
    store(_NCH - 2, 0)
    wait(1)
    store(_NCH - 1, 1)

  return gather


_gather_by_d = {}


def _sc_gather_call(table, idx, D):
  key = (D, table.dtype)
  if key not in _gather_by_d:
    _gather_by_d[key] = _make_sc_gather(D, table.dtype)
  return _gather_by_d[key](table, idx)


_BLK = 512
_NSET = 4


def _ptm0_body(fea_ref, w_ref, st_ref):
  raw = jnp.dot(fea_ref[...], w_ref[...], preferred_element_type=_F32)
  s1 = jnp.sum(raw, axis=0, keepdims=True)
  s2 = jnp.sum(raw * raw, axis=0, keepdims=True)
  c4 = w_ref.shape[1]
  st_ref[...] = jnp.concatenate([s1, s2, jnp.zeros((6, c4), _F32)], axis=0)


def _ptm0_call(feaT, w, c4):
  return pl.pallas_call(
      _ptm0_body,
      out_shape=jax.ShapeDtypeStruct((8, c4), _F32),
  )(feaT, w)


def _agg(infl3, neigh, s0, Din):
  nf = [neigh[k, :, :Din].astype(_F32) for k in range(K)]
  aggs = []
  for s in range(KS):
    acc = infl3[0, :, s0 + s:s0 + s + 1] * nf[0]
    for k in range(1, K):
      acc = acc + infl3[k, :, s0 + s:s0 + s + 1] * nf[k]
    aggs.append(acc)
  return jnp.concatenate(aggs, axis=1)


def _stats_update(o, st_ref, dout):
  @pl.when(pl.program_id(0) == 0)
  def _():
    st_ref[...] = jnp.zeros_like(st_ref)

  s1 = jnp.sum(o, axis=0, keepdims=True)
  s2 = jnp.sum(o * o, axis=0, keepdims=True)
  st_ref[...] += jnp.concatenate([s1, s2, jnp.zeros((6, dout), _F32)], axis=0)


def _body0_body(g1x_ref, xyz_ref, kpt_ref, kpsq_ref, w_ref,
                o_ref, st_ref, infl_ref):
  blk = g1x_ref.shape[1]
  gxyz = g1x_ref[:, :, CIN:CIN + 16]
  cx = xyz_ref[...].reshape(1, blk, 16)
  rel = (gxyz - cx) * (1.0 / RADIUS)
  rel2 = rel.reshape(K * blk, 16)
  ones = jnp.ones((16, 8 * _NSET), _F32)
  r2 = jnp.dot(rel2 * rel2, ones, preferred_element_type=_F32)
  rk = jnp.dot(rel2, kpt_ref[...], preferred_element_type=_F32)
  d2 = r2 - 2.0 * rk + kpsq_ref[...]
  dist = jnp.sqrt(d2 + 1e-12)
  infl = jnp.maximum(1.0 - dist, 0.0)
  infl3 = infl.reshape(K, blk, 8 * _NSET)
  infl_ref[...] = infl3.astype(_BF16)

  agg = _agg(infl3, g1x_ref[...], 8 * 2, CIN)
  o = jnp.dot(agg, w_ref[...], preferred_element_type=_F32)
  o_ref[...] = o
  _stats_update(o, st_ref, COUT)


def _body0_call(g1x, xyzp, kpt, kpsq, wf):
  grid = (M // _BLK,)
  return pl.pallas_call(
      _body0_body,
      grid=grid,
      in_specs=[
          pl.BlockSpec((K, _BLK, 2 * CIN), lambda i: (0, i, 0)),
          pl.BlockSpec((_BLK, 16), lambda i: (i, 0)),
          pl.BlockSpec((16, 8 * _NSET), lambda i: (0, 0)),
          pl.BlockSpec((1, 8 * _NSET), lambda i: (0, 0)),
          pl.BlockSpec((KS * CIN, COUT), lambda i: (0, 0)),
      ],
      out_specs=(
          pl.BlockSpec((_BLK, COUT), lambda i: (i, 0)),
          pl.BlockSpec((8, COUT), lambda i: (0, 0)),
          pl.BlockSpec((K, _BLK, 8 * _NSET), lambda i: (0, i, 0)),
      ),
      out_shape=(
          jax.ShapeDtypeStruct((M, COUT), _F32),
          jax.ShapeDtypeStruct((8, COUT), _F32),
          jax.ShapeDtypeStruct((K, M, 8 * _NSET), _BF16),
      ),
  )(g1x, xyzp, kpt, kpsq, wf)


def _ptm1_body(g1x_ref, infl_ref, w0_ref, st0_ref, g0_ref, b0_ref, w1_ref,
               o_ref, st_ref):
  c4 = w0_ref.shape[1]
  st0 = st0_ref[...]
  mean = st0[0:1, :] / M
  var = st0[1:2, :] / M - mean * mean
  inv = lax.rsqrt(var + EPS)
  a = inv * g0_ref[...]
  c = b0_ref[...] - mean * a
  infl3 = infl_ref[...].astype(_F32)
  nf = []
  for k in range(K):
    raw = jnp.dot(g1x_ref[k, :, :CIN], w0_ref[...],
                  preferred_element_type=_F32)
    nf.append(jnp.maximum(raw * a + c, 0.0))
  aggs = []
  for s in range(KS):
    acc = infl3[0, :, s:s + 1] * nf[0]
    for k in range(1, K):
      acc = acc + infl3[k, :, s:s + 1] * nf[k]
    aggs.append(acc)
  agg = jnp.concatenate(aggs, axis=1)
  o = jnp.dot(agg, w1_ref[...], preferred_element_type=_F32)
  o_ref[...] = o
  _stats_update(o, st_ref, c4)


def _ptm1_call(g1x, infl4, w0, st0, g0, b0, w1f, c4):
  grid = (M // _BLK,)
  return pl.pallas_call(
      _ptm1_body,
      grid=grid,
      in_specs=[
          pl.BlockSpec((K, _BLK, 2 * CIN), lambda i: (0, i, 0)),
          pl.BlockSpec((K, _BLK, 8 * _NSET), lambda i: (0, i, 0)),
          pl.BlockSpec((CIN, c4), lambda i: (0, 0)),
          pl.BlockSpec((8, c4), lambda i: (0, 0)),
          pl.BlockSpec((1, c4), lambda i: (0, 0)),
          pl.BlockSpec((1, c4), lambda i: (0, 0)),
          pl.BlockSpec((KS * c4, c4), lambda i: (0, 0)),
      ],
      out_specs=(
          pl.BlockSpec((_BLK, c4), lambda i: (i, 0)),
          pl.BlockSpec((8, c4), lambda i: (0, 0)),
      ),
      out_shape=(
          jax.ShapeDtypeStruct((M, c4), _F32),
          jax.ShapeDtypeStruct((8, c4), _F32),
      ),
  )(g1x, infl4, w0, st0, g0.reshape(1, c4), b0.reshape(1, c4), w1f)


def _kpconv_body(neigh_ref, infl_ref, w_ref, o_ref, st_ref, *, Din, Dout, s0):
  agg = _agg(infl_ref[...].astype(_F32), neigh_ref[...], s0, Din)
  o = jnp.dot(agg, w_ref[...], preferred_element_type=_F32)
  o_ref[...] = o
  _stats_update(o, st_ref, Dout)


def _kpconv_call(neigh, infl4, wf, Din, Dout, s0, blk=1024):
  grid = (M // blk,)
  dblk = neigh.shape[2]
  return pl.pallas_call(
      functools.partial(_kpconv_body, Din=Din, Dout=Dout, s0=s0),
      grid=grid,
      in_specs=[
          pl.BlockSpec((K, blk, dblk), lambda i: (0, i, 0)),
          pl.BlockSpec((K, blk, 8 * _NSET), lambda i: (0, i, 0)),
          pl.BlockSpec((KS * Din, Dout), lambda i: (0, 0)),
      ],
      out_specs=(
          pl.BlockSpec((blk, Dout), lambda i: (i, 0)),
          pl.BlockSpec((8, Dout), lambda i: (0, 0)),
      ),
      out_shape=(
          jax.ShapeDtypeStruct((M, Dout), _F32),
          jax.ShapeDtypeStruct((8, Dout), _F32),
      ),
  )(neigh, infl4, wf)


def _ptm2_body(neigh_ref, infl_ref, w_ref, pm_ref, *, Din, s0):
  agg = _agg(infl_ref[...].astype(_F32), neigh_ref[...], s0, Din)
  o = jnp.dot(agg, w_ref[...], preferred_element_type=_F32)
  a0 = o[:, 0:1] / TAU
  a1 = o[:, 1:2] / TAU
  mx = jnp.maximum(a0, a1)
  e0 = jnp.exp(a0 - mx)
  e1 = jnp.exp(a1 - mx)
  pm = e1 / (e0 + e1)
  pm_ref[...] = jnp.broadcast_to(pm, (pm_ref.shape[0], COUT))


def _ptm2_call(neigh, infl4, wf, Din, s0, blk=1024):
  grid = (M // blk,)
  dblk = neigh.shape[2]
  return pl.pallas_call(
      functools.partial(_ptm2_body, Din=Din, s0=s0),
      grid=grid,
      in_specs=[
          pl.BlockSpec((K, blk, dblk), lambda i: (0, i, 0)),
          pl.BlockSpec((K, blk, 8 * _NSET), lambda i: (0, i, 0)),
          pl.BlockSpec((KS * Din, 8), lambda i: (0, 0)),
      ],
      out_specs=pl.BlockSpec((blk, COUT), lambda i: (i, 0)),
      out_shape=jax.ShapeDtypeStruct((M, COUT), _F32),
  )(neigh, infl4, wf)


def _bn_from_stats(st, g, b):
  mean = st[0:1, :] / M
  var = st[1:2, :] / M - mean * mean
  inv = lax.rsqrt(var + EPS)
  a = inv * g
  return a, b - mean * a


def _fin_p1_body(raw_ref, st_ref, g_ref, b_ref, out_ref):
  a, c = _bn_from_stats(st_ref[...], g_ref[...], b_ref[...])
  y = jnp.maximum(raw_ref[...] * a + c, 0.0)
  d = raw_ref.shape[1]
  out_ref[...] = jnp.concatenate(
      [y, jnp.zeros((raw_ref.shape[0], COUT - d), _F32)], axis=1)


def _fin_p1_call(raw, st, g, b):
  d = raw.shape[1]
  return pl.pallas_call(
      _fin_p1_body,
      out_shape=jax.ShapeDtypeStruct((M, COUT), _F32),
  )(raw, st, g.reshape(1, d), b.reshape(1, d))


def _chw(m0, m1):
  a0 = m0 / TAU
  a1 = m1 / TAU
  mx = jnp.maximum(a0, a1)
  e0 = jnp.exp(a0 - mx)
  e1 = jnp.exp(a1 - mx)
  s = e0 + e1
  return e0 / s, e1 / s


def _fin_o0_body(raw_ref, st_ref, g_ref, b_ref, pm_ref, m0_ref, m1_ref,
                 out_ref):
  a, c = _bn_from_stats(st_ref[...], g_ref[...], b_ref[...])
  w0, w1 = _chw(m0_ref[...], m1_ref[...])
  o = jnp.maximum(raw_ref[...] * a + c, 0.0)
  out_ref[...] = o * (pm_ref[...] * w1 + w0)


def _fin_o0_call(raw, st, g, b, pm, m0, m1):
  return pl.pallas_call(
      _fin_o0_body,
      out_shape=jax.ShapeDtypeStruct((M, COUT), _F32),
  )(raw, st, g.reshape(1, COUT), b.reshape(1, COUT), pm,
    m0.reshape(1, COUT), m1.reshape(1, COUT))


_FLOP_SCALE0 = float(K * (CIN + 1))
_FLOP_SCALE1 = float(K * (COUT + 1))


def _tail_body(o0_ref, o1raw_ref, st1_ref, g1_ref, b1_ref, pm_ref,
               m0l0_ref, m1l0_ref, m0l1_ref, m1l1_ref,
               tw_ref, tb_ref, tg_ref, tbb_ref, fea_ref,
               out_ref, f0_ref, f1_ref):
  a, c = _bn_from_stats(st1_ref[...], g1_ref[...], b1_ref[...])
  w0l1, w1l1 = _chw(m0l1_ref[...], m1l1_ref[...])
  pm = pm_ref[...]
  o1 = jnp.maximum(o1raw_ref[...] * a + c, 0.0) * (pm * w1l1 + w0l1)
  cat = jnp.concatenate([o0_ref[...], o1], axis=1)
  traw = jnp.dot(cat, tw_ref[...], preferred_element_type=_F32) + tb_ref[...]
  s1 = jnp.sum(traw, axis=0, keepdims=True)
  s2 = jnp.sum(traw * traw, axis=0, keepdims=True)
  mean = s1 / M
  var = s2 / M - mean * mean
  inv = lax.rsqrt(var + EPS)
  t = (traw - mean) * (inv * tg_ref[...]) + tbb_ref[...]
  out_ref[...] = jnp.maximum(t + fea_ref[...], 0.0)
  w0l0, w1l0 = _chw(m0l0_ref[...], m1l0_ref[...])
  f0_ref[...] = (pm * w1l0 + w0l0) * _FLOP_SCALE0
  f1_ref[...] = (pm * w1l1 + w0l1) * _FLOP_SCALE1


def _tail_call(o0, o1raw, st1, g1, b1, pm, m0l0, m1l0, m0l1, m1l1,
               tw, tb, tg, tbb, feaT):
  r = lambda v: v.reshape(1, COUT)
  return pl.pallas_call(
      _tail_body,
      out_shape=(
          jax.ShapeDtypeStruct((M, COUT), _F32),
          jax.ShapeDtypeStruct((M, COUT), _F32),
          jax.ShapeDtypeStruct((M, COUT), _F32),
      ),
  )(o0, o1raw, st1, r(g1), r(b1), pm, r(m0l0), r(m1l0), r(m0l1), r(m1l1),
    tw, r(tb), r(tg), r(tbb), feaT)


def _kp_prep(kps):
  kpt = jnp.zeros((16, 8 * _NSET), _F32)
  kpsq = jnp.zeros((1, 8 * _NSET), _F32)
  for i, kp in enumerate(kps):
    kpt = kpt.at[:3, 8 * i:8 * i + KS].set(jnp.transpose(kp))
    kpsq = kpsq.at[0, 8 * i:8 * i + KS].set(jnp.sum(kp * kp, axis=1))
  return kpt, kpsq


_TOTAL_FLOPS = float(B * N * K * COUT * ((CIN + 1) + (COUT + 1)))


def kernel(xyz, fea, knn_idx, ch_mask, body0_kp, body0_W, body0_g, body0_b,
           body1_kp, body1_W, body1_g, body1_b, ptm0_W, ptm0_g, ptm0_b,
           ptm1_kp, ptm1_W, ptm1_g, ptm1_b, ptm2_kp, ptm2_W, tail_W,
           tail_bias, tail_g, tail_bb):
  c4 = ptm0_W.shape[1]
  feaT = fea.transpose(0, 2, 1).reshape(M, CIN)
  xyzt = xyz.transpose(0, 2, 1).reshape(M, 3)
  xyzp = jnp.concatenate([xyzt, jnp.zeros((M, 13), _F32)], axis=1)
  table1 = jnp.concatenate([feaT, xyzp, jnp.zeros((M, CIN - 16), _F32)],
                           axis=1)
  idxf = (knn_idx.astype(jnp.int32)
          + (jnp.arange(B, dtype=jnp.int32) * N)[:, None, None]
          ).transpose(2, 0, 1).reshape(MK)

  kpt, kpsq = _kp_prep([ptm1_kp, ptm2_kp, body0_kp, body1_kp])

  g1x = _sc_gather_call(table1, idxf, 2 * CIN).reshape(K, M, 2 * CIN)
  o0raw, st_o0, infl4 = _body0_call(
      g1x, xyzp, kpt, kpsq, body0_W.reshape(KS * CIN, COUT))

  st_p0 = _ptm0_call(feaT, ptm0_W, c4)
  p1raw, st_p1 = _ptm1_call(g1x, infl4, ptm0_W, st_p0, ptm0_g, ptm0_b,
                            ptm1_W.reshape(KS * c4, c4), c4)
  p1 = _fin_p1_call(p1raw, st_p1, ptm1_g, ptm1_b)
  g3 = _sc_gather_call(p1, idxf, COUT).reshape(K, M, COUT)
  w2p = jnp.zeros((KS * c4, 8), _F32).at[:, :2].set(ptm2_W.reshape(KS * c4, 2))
  pm = _ptm2_call(g3, infl4, w2p, c4, 8 * 1)

  m0l0, m1l0 = ch_mask[0, :, 0], ch_mask[0, :, 1]
  o0 = _fin_o0_call(o0raw, st_o0, body0_g, body0_b, pm, m0l0, m1l0)

  g4 = _sc_gather_call(o0, idxf, COUT).reshape(K, M, COUT)
  o1raw, st_o1 = _kpconv_call(g4, infl4, body1_W.reshape(KS * COUT, COUT),
                              COUT, COUT, 8 * 3)

  m0l1, m1l1 = ch_mask[1, :, 0], ch_mask[1, :, 1]
  out_pm, f0, f1 = _tail_call(o0, o1raw, st_o1, body1_g, body1_b, pm,
                              m0l0, m1l0, m0l1, m1l1,
                              tail_W, tail_bias, tail_g, tail_bb, feaT)

  out = out_pm.reshape(B, N, COUT).transpose(0, 2, 1)
  flops = jnp.concatenate([
      f0.reshape(B, N, COUT).transpose(0, 2, 1).reshape(-1),
      f1.reshape(B, N, COUT).transpose(0, 2, 1).reshape(-1),
  ], 0)
  return (out, flops, jnp.float32(_TOTAL_FLOPS))


# speedup vs baseline: 1.0230x; 1.0230x over previous
import functools

import jax
import jax.numpy as jnp
from jax import lax
from jax.experimental import pallas as pl
from jax.experimental.pallas import tpu as pltpu
from jax.experimental.pallas import tpu_sc as plsc

B, N, K, CIN, COUT, KS = 2, 4096, 16, 128, 128, 5
M = B * N
MK = M * K
RADIUS, TAU, EPS = 1.0, 1.0, 1e-5
_F32 = jnp.float32
_BF16 = jnp.bfloat16

_NC, _NS = 2, 16
_NW = _NC * _NS
_PW = MK // _NW
_CH = 128
_NCH = _PW // _CH


def _make_sc_gather(D, dtype):
  mesh = plsc.VectorSubcoreMesh(core_axis_name="c", subcore_axis_name="s")

  @functools.partial(
      pl.kernel,
      out_type=jax.ShapeDtypeStruct((MK, D), dtype),
      mesh=mesh,
      scratch_types=[
          pltpu.VMEM((_PW,), jnp.int32),
          pltpu.VMEM((_CH, D), dtype),
          pltpu.VMEM((_CH, D), dtype),
          pltpu.SemaphoreType.DMA,
          pltpu.SemaphoreType.DMA,
      ],
  )
  def gather(table_hbm, idx_hbm, out_hbm, idx_v, buf0, buf1, sem0, sem1):
    wid = lax.axis_index("s") * _NC + lax.axis_index("c")
    base = wid * _PW
    pltpu.sync_copy(idx_hbm.at[pl.ds(base, _PW)], idx_v)
    bufs = (buf0, buf1)
    sems = (sem0, sem1)

    def start(c, b):
      pltpu.make_async_copy(
          table_hbm.at[idx_v.at[pl.ds(c * _CH, _CH)]], bufs[b], sems[b]
      ).start()

    def wait(b):
      pltpu.make_async_copy(
          table_hbm.at[idx_v.at[pl.ds(0, _CH)]], bufs[b], sems[b]
      ).wait()

    def store(c, b):
      pltpu.sync_copy(bufs[b], out_hbm.at[pl.ds(base + c * _CH, _CH)])

    start(0, 0)
    start(1, 1)

    def body(j, carry):
      c0 = 2 * j
      wait(0)
      store(c0, 0)
      start(c0 + 2, 0)
      wait(1)
      store(c0 + 1, 1)
      start(c0 + 3, 1)
      return carry

    lax.fori_loop(0, _NCH // 2 - 1, body, 0)
    wait(0)
    store(_NCH - 2, 0)
    wait(1)
    store(_NCH - 1, 1)

  return gather


_gather_by_d = {}


def _sc_gather_call(table, idx, D):
  key = (D, table.dtype)
  if key not in _gather_by_d:
    _gather_by_d[key] = _make_sc_gather(D, table.dtype)
  return _gather_by_d[key](table, idx)


_BLK = 512
_NSET = 4


def _ptm0_body(fea_ref, w_ref, st_ref):
  raw = jnp.dot(fea_ref[...], w_ref[...], preferred_element_type=_F32)
  s1 = jnp.sum(raw, axis=0, keepdims=True)
  s2 = jnp.sum(raw * raw, axis=0, keepdims=True)
  c4 = w_ref.shape[1]
  st_ref[...] = jnp.concatenate([s1, s2, jnp.zeros((6, c4), _F32)], axis=0)


def _ptm0_call(feaT, w, c4):
  return pl.pallas_call(
      _ptm0_body,
      out_shape=jax.ShapeDtypeStruct((8, c4), _F32),
  )(feaT, w)


def _agg(infl3, neigh, s0, Din):
  nf = [neigh[k, :, :Din].astype(_F32) for k in range(K)]
  aggs = []
  for s in range(KS):
    acc = infl3[0, :, s0 + s:s0 + s + 1] * nf[0]
    for k in range(1, K):
      acc = acc + infl3[k, :, s0 + s:s0 + s + 1] * nf[k]
    aggs.append(acc)
  return jnp.concatenate(aggs, axis=1)


def _stats_update(o, st_ref, dout):
  @pl.when(pl.program_id(0) == 0)
  def _():
    st_ref[...] = jnp.zeros_like(st_ref)

  s1 = jnp.sum(o, axis=0, keepdims=True)
  s2 = jnp.sum(o * o, axis=0, keepdims=True)
  st_ref[...] += jnp.concatenate([s1, s2, jnp.zeros((6, dout), _F32)], axis=0)


def _body0_body(g1x_ref, xyz_ref, kpt_ref, kpsq_ref, w_ref,
                o_ref, st_ref, infl_ref):
  blk = g1x_ref.shape[1]
  gxyz = g1x_ref[:, :, CIN:CIN + 16]
  cx = xyz_ref[...].reshape(1, blk, 16)
  rel = (gxyz - cx) * (1.0 / RADIUS)
  rel2 = rel.reshape(K * blk, 16)
  ones = jnp.ones((16, 8 * _NSET), _F32)
  r2 = jnp.dot(rel2 * rel2, ones, preferred_element_type=_F32)
  rk = jnp.dot(rel2, kpt_ref[...], preferred_element_type=_F32)
  d2 = r2 - 2.0 * rk + kpsq_ref[...]
  dist = jnp.sqrt(d2 + 1e-12)
  infl = jnp.maximum(1.0 - dist, 0.0)
  infl3 = infl.reshape(K, blk, 8 * _NSET)
  infl_ref[...] = infl3.astype(_BF16)

  agg = _agg(infl3, g1x_ref[...], 8 * 2, CIN)
  o = jnp.dot(agg, w_ref[...], preferred_element_type=_F32)
  o_ref[...] = o
  _stats_update(o, st_ref, COUT)


def _body0_call(g1x, xyzp, kpt, kpsq, wf):
  grid = (M // _BLK,)
  return pl.pallas_call(
      _body0_body,
      grid=grid,
      in_specs=[
          pl.BlockSpec((K, _BLK, 2 * CIN), lambda i: (0, i, 0)),
          pl.BlockSpec((_BLK, 16), lambda i: (i, 0)),
          pl.BlockSpec((16, 8 * _NSET), lambda i: (0, 0)),
          pl.BlockSpec((1, 8 * _NSET), lambda i: (0, 0)),
          pl.BlockSpec((KS * CIN, COUT), lambda i: (0, 0)),
      ],
      out_specs=(
          pl.BlockSpec((_BLK, COUT), lambda i: (i, 0)),
          pl.BlockSpec((8, COUT), lambda i: (0, 0)),
          pl.BlockSpec((K, _BLK, 8 * _NSET), lambda i: (0, i, 0)),
      ),
      out_shape=(
          jax.ShapeDtypeStruct((M, COUT), _F32),
          jax.ShapeDtypeStruct((8, COUT), _F32),
          jax.ShapeDtypeStruct((K, M, 8 * _NSET), _BF16),
      ),
  )(g1x, xyzp, kpt, kpsq, wf)


def _ptm1_body(g1x_ref, infl_ref, w0_ref, st0_ref, g0_ref, b0_ref, w1_ref,
               o_ref, st_ref):
  c4 = w0_ref.shape[1]
  st0 = st0_ref[...]
  mean = st0[0:1, :] / M
  var = st0[1:2, :] / M - mean * mean
  inv = lax.rsqrt(var + EPS)
  a = inv * g0_ref[...]
  c = b0_ref[...] - mean * a
  infl3 = infl_ref[...].astype(_F32)
  nf = []
  for k in range(K):
    raw = jnp.dot(g1x_ref[k, :, :CIN], w0_ref[...],
                  preferred_element_type=_F32)
    nf.append(jnp.maximum(raw * a + c, 0.0))
  aggs = []
  for s in range(KS):
    acc = infl3[0, :, s:s + 1] * nf[0]
    for k in range(1, K):
      acc = acc + infl3[k, :, s:s + 1] * nf[k]
    aggs.append(acc)
  agg = jnp.concatenate(aggs, axis=1)
  o = jnp.dot(agg, w1_ref[...], preferred_element_type=_F32)
  o_ref[...] = o
  _stats_update(o, st_ref, c4)


def _ptm1_call(g1x, infl4, w0, st0, g0, b0, w1f, c4):
  grid = (M // _BLK,)
  return pl.pallas_call(
      _ptm1_body,
      grid=grid,
      in_specs=[
          pl.BlockSpec((K, _BLK, 2 * CIN), lambda i: (0, i, 0)),
          pl.BlockSpec((K, _BLK, 8 * _NSET), lambda i: (0, i, 0)),
          pl.BlockSpec((CIN, c4), lambda i: (0, 0)),
          pl.BlockSpec((8, c4), lambda i: (0, 0)),
          pl.BlockSpec((1, c4), lambda i: (0, 0)),
          pl.BlockSpec((1, c4), lambda i: (0, 0)),
          pl.BlockSpec((KS * c4, c4), lambda i: (0, 0)),
      ],
      out_specs=(
          pl.BlockSpec((_BLK, c4), lambda i: (i, 0)),
          pl.BlockSpec((8, c4), lambda i: (0, 0)),
      ),
      out_shape=(
          jax.ShapeDtypeStruct((M, c4), _F32),
          jax.ShapeDtypeStruct((8, c4), _F32),
      ),
  )(g1x, infl4, w0, st0, g0.reshape(1, c4), b0.reshape(1, c4), w1f)


def _kpconv_body(neigh_ref, infl_ref, w_ref, o_ref, st_ref, *, Din, Dout, s0):
  agg = _agg(infl_ref[...].astype(_F32), neigh_ref[...], s0, Din)
  o = jnp.dot(agg, w_ref[...], preferred_element_type=_F32)
  o_ref[...] = o
  _stats_update(o, st_ref, Dout)


def _kpconv_call(neigh, infl4, wf, Din, Dout, s0, blk=_BLK):
  grid = (M // blk,)
  dblk = neigh.shape[2]
  return pl.pallas_call(
      functools.partial(_kpconv_body, Din=Din, Dout=Dout, s0=s0),
      grid=grid,
      in_specs=[
          pl.BlockSpec((K, blk, dblk), lambda i: (0, i, 0)),
          pl.BlockSpec((K, blk, 8 * _NSET), lambda i: (0, i, 0)),
          pl.BlockSpec((KS * Din, Dout), lambda i: (0, 0)),
      ],
      out_specs=(
          pl.BlockSpec((blk, Dout), lambda i: (i, 0)),
          pl.BlockSpec((8, Dout), lambda i: (0, 0)),
      ),
      out_shape=(
          jax.ShapeDtypeStruct((M, Dout), _F32),
          jax.ShapeDtypeStruct((8, Dout), _F32),
      ),
  )(neigh, infl4, wf)


def _ptm2_body(neigh_ref, infl_ref, w_ref, pm_ref, *, Din, s0):
  agg = _agg(infl_ref[...].astype(_F32), neigh_ref[...], s0, Din)
  o = jnp.dot(agg, w_ref[...], preferred_element_type=_F32)
  a0 = o[:, 0:1] / TAU
  a1 = o[:, 1:2] / TAU
  mx = jnp.maximum(a0, a1)
  e0 = jnp.exp(a0 - mx)
  e1 = jnp.exp(a1 - mx)
  pm = e1 / (e0 + e1)
  pm_ref[...] = jnp.broadcast_to(pm, (pm_ref.shape[0], COUT))


def _ptm2_call(neigh, infl4, wf, Din, s0, blk=_BLK):
  grid = (M // blk,)
  dblk = neigh.shape[2]
  return pl.pallas_call(
      functools.partial(_ptm2_body, Din=Din, s0=s0),
      grid=grid,
      in_specs=[
          pl.BlockSpec((K, blk, dblk), lambda i: (0, i, 0)),
          pl.BlockSpec((K, blk, 8 * _NSET), lambda i: (0, i, 0)),
          pl.BlockSpec((KS * Din, 8), lambda i: (0, 0)),
      ],
      out_specs=pl.BlockSpec((blk, COUT), lambda i: (i, 0)),
      out_shape=jax.ShapeDtypeStruct((M, COUT), _F32),
  )(neigh, infl4, wf)


def _bn_from_stats(st, g, b):
  mean = st[0:1, :] / M
  var = st[1:2, :] / M - mean * mean
  inv = lax.rsqrt(var + EPS)
  a = inv * g
  return a, b - mean * a


def _fin_p1_body(raw_ref, st_ref, g_ref, b_ref, out_ref):
  a, c = _bn_from_stats(st_ref[...], g_ref[...], b_ref[...])
  y = jnp.maximum(raw_ref[...] * a + c, 0.0)
  d = raw_ref.shape[1]
  out_ref[...] = jnp.concatenate(
      [y, jnp.zeros((raw_ref.shape[0], COUT - d), _F32)], axis=1)


def _fin_p1_call(raw, st, g, b):
  d = raw.shape[1]
  return pl.pallas_call(
      _fin_p1_body,
      out_shape=jax.ShapeDtypeStruct((M, COUT), _F32),
  )(raw, st, g.reshape(1, d), b.reshape(1, d))


def _chw(m0, m1):
  a0 = m0 / TAU
  a1 = m1 / TAU
  mx = jnp.maximum(a0, a1)
  e0 = jnp.exp(a0 - mx)
  e1 = jnp.exp(a1 - mx)
  s = e0 + e1
  return e0 / s, e1 / s


def _fin_o0_body(raw_ref, st_ref, g_ref, b_ref, pm_ref, m0_ref, m1_ref,
                 out_ref):
  a, c = _bn_from_stats(st_ref[...], g_ref[...], b_ref[...])
  w0, w1 = _chw(m0_ref[...], m1_ref[...])
  o = jnp.maximum(raw_ref[...] * a + c, 0.0)
  out_ref[...] = o * (pm_ref[...] * w1 + w0)


def _fin_o0_call(raw, st, g, b, pm, m0, m1):
  return pl.pallas_call(
      _fin_o0_body,
      out_shape=jax.ShapeDtypeStruct((M, COUT), _F32),
  )(raw, st, g.reshape(1, COUT), b.reshape(1, COUT), pm,
    m0.reshape(1, COUT), m1.reshape(1, COUT))


_FLOP_SCALE0 = float(K * (CIN + 1))
_FLOP_SCALE1 = float(K * (COUT + 1))


def _tail_body(o0_ref, o1raw_ref, st1_ref, g1_ref, b1_ref, pm_ref,
               m0l0_ref, m1l0_ref, m0l1_ref, m1l1_ref,
               tw_ref, tb_ref, tg_ref, tbb_ref, fea_ref,
               out_ref, f0_ref, f1_ref):
  a, c = _bn_from_stats(st1_ref[...], g1_ref[...], b1_ref[...])
  w0l1, w1l1 = _chw(m0l1_ref[...], m1l1_ref[...])
  pm = pm_ref[...]
  o1 = jnp.maximum(o1raw_ref[...] * a + c, 0.0) * (pm * w1l1 + w0l1)
  cat = jnp.concatenate([o0_ref[...], o1], axis=1)
  traw = jnp.dot(cat, tw_ref[...], preferred_element_type=_F32) + tb_ref[...]
  s1 = jnp.sum(traw, axis=0, keepdims=True)
  s2 = jnp.sum(traw * traw, axis=0, keepdims=True)
  mean = s1 / M
  var = s2 / M - mean * mean
  inv = lax.rsqrt(var + EPS)
  t = (traw - mean) * (inv * tg_ref[...]) + tbb_ref[...]
  out_ref[...] = jnp.maximum(t + fea_ref[...], 0.0)
  w0l0, w1l0 = _chw(m0l0_ref[...], m1l0_ref[...])
  f0_ref[...] = (pm * w1l0 + w0l0) * _FLOP_SCALE0
  f1_ref[...] = (pm * w1l1 + w0l1) * _FLOP_SCALE1


def _tail_call(o0, o1raw, st1, g1, b1, pm, m0l0, m1l0, m0l1, m1l1,
               tw, tb, tg, tbb, feaT):
  r = lambda v: v.reshape(1, COUT)
  return pl.pallas_call(
      _tail_body,
      out_shape=(
          jax.ShapeDtypeStruct((M, COUT), _F32),
          jax.ShapeDtypeStruct((M, COUT), _F32),
          jax.ShapeDtypeStruct((M, COUT), _F32),
      ),
  )(o0, o1raw, st1, r(g1), r(b1), pm, r(m0l0), r(m1l0), r(m0l1), r(m1l1),
    tw, r(tb), r(tg), r(tbb), feaT)


def _kp_prep(kps):
  kpt = jnp.zeros((16, 8 * _NSET), _F32)
  kpsq = jnp.zeros((1, 8 * _NSET), _F32)
  for i, kp in enumerate(kps):
    kpt = kpt.at[:3, 8 * i:8 * i + KS].set(jnp.transpose(kp))
    kpsq = kpsq.at[0, 8 * i:8 * i + KS].set(jnp.sum(kp * kp, axis=1))
  return kpt, kpsq


_TOTAL_FLOPS = float(B * N * K * COUT * ((CIN + 1) + (COUT + 1)))


def kernel(xyz, fea, knn_idx, ch_mask, body0_kp, body0_W, body0_g, body0_b,
           body1_kp, body1_W, body1_g, body1_b, ptm0_W, ptm0_g, ptm0_b,
           ptm1_kp, ptm1_W, ptm1_g, ptm1_b, ptm2_kp, ptm2_W, tail_W,
           tail_bias, tail_g, tail_bb):
  c4 = ptm0_W.shape[1]
  feaT = fea.transpose(0, 2, 1).reshape(M, CIN)
  xyzt = xyz.transpose(0, 2, 1).reshape(M, 3)
  xyzp = jnp.concatenate([xyzt, jnp.zeros((M, 13), _F32)], axis=1)
  table1 = jnp.concatenate([feaT, xyzp, jnp.zeros((M, CIN - 16), _F32)],
                           axis=1)
  idxf = (knn_idx.astype(jnp.int32)
          + (jnp.arange(B, dtype=jnp.int32) * N)[:, None, None]
          ).transpose(2, 0, 1).reshape(MK)

  kpt, kpsq = _kp_prep([ptm1_kp, ptm2_kp, body0_kp, body1_kp])

  g1x = _sc_gather_call(table1, idxf, 2 * CIN).reshape(K, M, 2 * CIN)
  o0raw, st_o0, infl4 = _body0_call(
      g1x, xyzp, kpt, kpsq, body0_W.reshape(KS * CIN, COUT))

  st_p0 = _ptm0_call(feaT, ptm0_W, c4)
  p1raw, st_p1 = _ptm1_call(g1x, infl4, ptm0_W, st_p0, ptm0_g, ptm0_b,
                            ptm1_W.reshape(KS * c4, c4), c4)
  p1 = _fin_p1_call(p1raw, st_p1, ptm1_g, ptm1_b)
  g3 = _sc_gather_call(p1, idxf, COUT).reshape(K, M, COUT)
  w2p = jnp.zeros((KS * c4, 8), _F32).at[:, :2].set(ptm2_W.reshape(KS * c4, 2))
  pm = _ptm2_call(g3, infl4, w2p, c4, 8 * 1)

  m0l0, m1l0 = ch_mask[0, :, 0], ch_mask[0, :, 1]
  o0 = _fin_o0_call(o0raw, st_o0, body0_g, body0_b, pm, m0l0, m1l0)

  g4 = _sc_gather_call(o0, idxf, COUT).reshape(K, M, COUT)
  o1raw, st_o1 = _kpconv_call(g4, infl4, body1_W.reshape(KS * COUT, COUT),
                              COUT, COUT, 8 * 3)

  m0l1, m1l1 = ch_mask[1, :, 0], ch_mask[1, :, 1]
  out_pm, f0, f1 = _tail_call(o0, o1raw, st_o1, body1_g, body1_b, pm,
                              m0l0, m1l0, m0l1, m1l1,
                              tail_W, tail_bias, tail_g, tail_bb, feaT)

  out = out_pm.reshape(B, N, COUT).transpose(0, 2, 1)
  flops = jnp.concatenate([
      f0.reshape(B, N, COUT).transpose(0, 2, 1).reshape(-1),
      f1.reshape(B, N, COUT).transpose(0, 2, 1).reshape(-1),
  ], 0)
  return (out, flops, jnp.float32(_TOTAL_FLOPS))
